# baseline (speedup 1.0000x reference)
import functools

import jax
import jax.numpy as jnp
from jax import lax
from jax.experimental import pallas as pl
from jax.experimental.pallas import tpu as pltpu
from jax.experimental.pallas import tpu_sc as plsc

K = 1024
ED = 256
B = 1024
BI = 8

_COMMIT = 0.25
_DIVERGE = 0.1


def _dist_body(y_ref, x_ref, wt_ref, close_ref, xl_ref):
    ib = pl.program_id(0)
    wt = wt_ref[...]
    xtb = x_ref[...].T
    kk = lax.broadcasted_iota(jnp.int32, (1, K), 1)
    for ii in range(BI):
        xcol = xtb[:, ii:ii + 1]
        xb = jnp.broadcast_to(xcol, (ED, K))
        diff = xb - wt
        sq = diff * diff
        rs = sq.reshape(2, 16, 8, K)
        acc = rs[:, 0]
        for j in range(1, 16):
            acc = acc + rs[:, j]
        v = ((acc[:, 0] + acc[:, 4]) + (acc[:, 2] + acc[:, 6])) + (
            (acc[:, 1] + acc[:, 5]) + (acc[:, 3] + acc[:, 7]))
        dsq = (v[0:1] + v[1:2])
        sd = dsq * lax.rsqrt(dsq)
        m = jnp.min(sd)
        idx = jnp.min(jnp.where(sd == m, kk, jnp.int32(2 ** 30)))
        dmin = jnp.min(dsq)
        yi = y_ref[ib * BI + ii]
        ind = (idx != yi).astype(jnp.float32)
        close_ref[ii:ii + 1, 0:1] = idx.reshape(1, 1)
        xl_ref[ii:ii + 1, 0:1] = (ind * dmin).reshape(1, 1)


def _perp_body(y2_ref, perp_ref):
    ycol = y2_ref[...]
    kk = lax.broadcasted_iota(jnp.int32, (1, K), 1)
    eq = (ycol == kk).astype(jnp.float32)
    counts = jnp.sum(eq, axis=0, keepdims=True)
    probs = counts * (1.0 / B)
    ent = jnp.sum(probs * jnp.log(probs + 1e-10))
    perp_ref[0:1, 0:1] = jnp.exp(-ent).reshape(1, 1)


def _combine_body(x_ref, wy_ref, xl_ref, qst_ref, loss_ref):
    xv = x_ref[...]
    q = wy_ref[...] - xv
    qst_ref[...] = xv + q
    qsum = jnp.sum(q * q)
    xlsum = jnp.sum(xl_ref[...])
    scale = 1.0 / (B * ED)
    loss_ref[0:1, 0:1] = ((1.0 + _COMMIT) * qsum * scale
                          - (1.0 + _DIVERGE) * xlsum * scale).reshape(1, 1)


_SC_NC = 2
_SC_NS = 16
_NW = _SC_NC * _SC_NS
_BPW = B // _NW


@functools.cache
def _gather_rows_kernel():
    mesh = plsc.VectorSubcoreMesh(core_axis_name="c", subcore_axis_name="s")

    @functools.partial(
        pl.kernel, mesh=mesh,
        out_type=jax.ShapeDtypeStruct((B, ED), jnp.float32),
        scratch_types=[
            pltpu.VMEM((_BPW,), jnp.int32),
            pltpu.VMEM((_BPW, ED), jnp.float32),
            pltpu.SemaphoreType.DMA,
        ],
    )
    def _gather_rows(table_hbm, idx_hbm, out_hbm, idx_v, rows_v, sem):
        wid = lax.axis_index("s") * _SC_NC + lax.axis_index("c")
        base = wid * _BPW
        pltpu.sync_copy(idx_hbm.at[pl.ds(base, _BPW)], idx_v)
        pltpu.async_copy(table_hbm.at[idx_v], rows_v, sem).wait()
        pltpu.sync_copy(rows_v, out_hbm.at[pl.ds(base, _BPW)])

    return _gather_rows


def kernel(x, y, W):
    y32 = y.astype(jnp.int32)
    wy = _gather_rows_kernel()(W, y32)

    wt = W.T
    close, xl = pl.pallas_call(
        _dist_body,
        grid=(B // BI,),
        in_specs=[
            pl.BlockSpec(memory_space=pltpu.SMEM),
            pl.BlockSpec((BI, ED), lambda ib: (ib, 0)),
            pl.BlockSpec((ED, K), lambda ib: (0, 0)),
        ],
        out_specs=[
            pl.BlockSpec((BI, 1), lambda ib: (ib, 0)),
            pl.BlockSpec((BI, 1), lambda ib: (ib, 0)),
        ],
        out_shape=[
            jax.ShapeDtypeStruct((B, 1), jnp.int32),
            jax.ShapeDtypeStruct((B, 1), jnp.float32),
        ],
    )(y32, x, wt)

    perp = pl.pallas_call(
        _perp_body,
        out_shape=jax.ShapeDtypeStruct((1, 1), jnp.float32),
    )(y32.reshape(B, 1))

    qst, loss = pl.pallas_call(
        _combine_body,
        out_shape=[
            jax.ShapeDtypeStruct((B, ED), jnp.float32),
            jax.ShapeDtypeStruct((1, 1), jnp.float32),
        ],
    )(x, wy, xl)

    return (loss.reshape(()), qst, perp.reshape(()), close)


# speedup vs baseline: 4.8112x; 4.8112x over previous
import functools

import jax
import jax.numpy as jnp
from jax import lax
from jax.experimental import pallas as pl
from jax.experimental.pallas import tpu as pltpu
from jax.experimental.pallas import tpu_sc as plsc

K = 1024
ED = 256
B = 1024
T = 8
BI = 128

_COMMIT = 0.25
_DIVERGE = 0.1
_BIGF = 3.0e38
_BIGI = 2 ** 30


def _exact_tree(sq):
    L = sq.shape[1]
    rs = sq.reshape(2, 16, 8, L)
    acc = rs[:, 0]
    for j in range(1, 16):
        acc = acc + rs[:, j]
    v = ((acc[:, 0] + acc[:, 4]) + (acc[:, 2] + acc[:, 6])) + (
        (acc[:, 1] + acc[:, 5]) + (acc[:, 3] + acc[:, 7]))
    return v[0:1] + v[1:2]


def _coarse_body(x_ref, wt_ref, cand_ref):
    xb = x_ref[...]
    wt = wt_ref[...]
    g = jnp.dot(xb, wt, precision=lax.Precision.HIGHEST,
                preferred_element_type=jnp.float32)
    wn = jnp.sum(wt * wt, axis=0, keepdims=True)
    d = wn - (g + g)
    kk = lax.broadcasted_iota(jnp.int32, (BI, K), 1)
    for t in range(T):
        m = jnp.min(d, axis=1, keepdims=True)
        it = jnp.min(jnp.where(d == m, kk, _BIGI), axis=1,
                     keepdims=True)
        cand_ref[t:t + 1, :] = it.T
        if t < T - 1:
            d = jnp.where(kk == it, _BIGF, d)


def _refine_body(y_ref, x_ref, wc_ref, cand_ref, close_ref, xl_ref):
    xt = x_ref[...].T
    dsq_rows = []
    for t in range(T):
        wct = wc_ref[t].T
        diff = xt - wct
        dsq_rows.append(_exact_tree(diff * diff))
    dsq = jnp.concatenate(dsq_rows, axis=0)
    sd = dsq * lax.rsqrt(dsq)
    cidx = cand_ref[...]
    m = jnp.min(sd, axis=0, keepdims=True)
    idx = jnp.min(jnp.where(sd == m, cidx, _BIGI), axis=0,
                  keepdims=True)
    dmin = jnp.min(dsq, axis=0, keepdims=True)
    yi = y_ref[0]
    ind = (idx != yi).astype(jnp.float32)
    close_ref[0] = idx
    xl_ref[0] = ind * dmin


def _perp_body(y2_ref, perp_ref):
    ycol = y2_ref[...]
    kk = lax.broadcasted_iota(jnp.int32, (1, K), 1)
    eq = (ycol == kk).astype(jnp.float32)
    counts = jnp.sum(eq, axis=0, keepdims=True)
    probs = counts * (1.0 / B)
    ent = jnp.sum(probs * jnp.log(probs + 1e-10))
    perp_ref[0:1, 0:1] = jnp.exp(-ent).reshape(1, 1)


def _combine_body(x_ref, wy_ref, xl_ref, qst_ref, loss_ref):
    xv = x_ref[...]
    q = wy_ref[...] - xv
    qst_ref[...] = xv + q
    qsum = jnp.sum(q * q)
    xlsum = jnp.sum(xl_ref[...])
    scale = 1.0 / (B * ED)
    loss_ref[0:1, 0:1] = ((1.0 + _COMMIT) * qsum * scale
                          - (1.0 + _DIVERGE) * xlsum * scale).reshape(1, 1)


_SC_NC = 2
_SC_NS = 16
_NW = _SC_NC * _SC_NS


@functools.cache
def _gather_rows_kernel(nrows):
    bpw = nrows // _NW
    mesh = plsc.VectorSubcoreMesh(core_axis_name="c", subcore_axis_name="s")

    @functools.partial(
        pl.kernel, mesh=mesh,
        out_type=jax.ShapeDtypeStruct((nrows, ED), jnp.float32),
        scratch_types=[
            pltpu.VMEM((bpw,), jnp.int32),
            pltpu.VMEM((bpw, ED), jnp.float32),
            pltpu.SemaphoreType.DMA,
        ],
    )
    def _gather_rows(table_hbm, idx_hbm, out_hbm, idx_v, rows_v, sem):
        wid = lax.axis_index("s") * _SC_NC + lax.axis_index("c")
        base = wid * bpw
        pltpu.sync_copy(idx_hbm.at[pl.ds(base, bpw)], idx_v)
        pltpu.async_copy(table_hbm.at[idx_v], rows_v, sem).wait()
        pltpu.sync_copy(rows_v, out_hbm.at[pl.ds(base, bpw)])

    return _gather_rows


def kernel(x, y, W):
    y32 = y.astype(jnp.int32)
    wy = _gather_rows_kernel(B)(W, y32)

    wt = W.T
    cand = pl.pallas_call(
        _coarse_body,
        grid=(B // BI,),
        in_specs=[
            pl.BlockSpec((BI, ED), lambda ib: (ib, 0)),
            pl.BlockSpec((ED, K), lambda ib: (0, 0)),
        ],
        out_specs=pl.BlockSpec((T, BI), lambda ib: (0, ib)),
        out_shape=jax.ShapeDtypeStruct((T, B), jnp.int32),
    )(x, wt)

    wc = _gather_rows_kernel(T * B)(W, cand.reshape(T * B))
    wc = wc.reshape(T, B, ED)

    close3, xl3 = pl.pallas_call(
        _refine_body,
        grid=(B // BI,),
        in_specs=[
            pl.BlockSpec((1, 1, BI), lambda ib: (ib, 0, 0)),
            pl.BlockSpec((BI, ED), lambda ib: (ib, 0)),
            pl.BlockSpec((T, BI, ED), lambda ib: (0, ib, 0)),
            pl.BlockSpec((T, BI), lambda ib: (0, ib)),
        ],
        out_specs=[
            pl.BlockSpec((1, 1, BI), lambda ib: (ib, 0, 0)),
            pl.BlockSpec((1, 1, BI), lambda ib: (ib, 0, 0)),
        ],
        out_shape=[
            jax.ShapeDtypeStruct((B // BI, 1, BI), jnp.int32),
            jax.ShapeDtypeStruct((B // BI, 1, BI), jnp.float32),
        ],
    )(y32.reshape(B // BI, 1, BI), x, wc, cand)
    close = close3.reshape(B, 1)

    perp = pl.pallas_call(
        _perp_body,
        out_shape=jax.ShapeDtypeStruct((1, 1), jnp.float32),
    )(y32.reshape(B, 1))

    qst, loss = pl.pallas_call(
        _combine_body,
        out_shape=[
            jax.ShapeDtypeStruct((B, ED), jnp.float32),
            jax.ShapeDtypeStruct((1, 1), jnp.float32),
        ],
    )(x, wy, xl3.reshape(B, 1))

    return (loss.reshape(()), qst, perp.reshape(()), close)


# speedup vs baseline: 5.7357x; 1.1922x over previous
import functools

import jax
import jax.numpy as jnp
from jax import lax
from jax.experimental import pallas as pl
from jax.experimental.pallas import tpu as pltpu
from jax.experimental.pallas import tpu_sc as plsc

K = 1024
ED = 256
B = 1024
T = 6
BI = 128
NB = B // BI

_COMMIT = 0.25
_DIVERGE = 0.1
_BIGF = 3.0e38
_BIGI = 2 ** 30


def _exact_tree(sq):
    L = sq.shape[1]
    rs = sq.reshape(2, 16, 8, L)
    acc = rs[:, 0]
    for j in range(1, 16):
        acc = acc + rs[:, j]
    v = ((acc[:, 0] + acc[:, 4]) + (acc[:, 2] + acc[:, 6])) + (
        (acc[:, 1] + acc[:, 5]) + (acc[:, 3] + acc[:, 7]))
    return v[0:1] + v[1:2]


def _coarse_body(y_ref, x_ref, w_ref, candy_ref):
    xt = x_ref[...].T
    w = w_ref[...]
    g = lax.dot_general(w, xt, (((1,), (0,)), ((), ())),
                        precision=lax.Precision.HIGHEST,
                        preferred_element_type=jnp.float32)
    wn = jnp.sum(w * w, axis=1, keepdims=True)
    d = wn - (g + g)
    kk = lax.broadcasted_iota(jnp.int32, (K, BI), 0)
    for t in range(T):
        m = jnp.min(d, axis=0, keepdims=True)
        it = jnp.min(jnp.where(d == m, kk, _BIGI), axis=0,
                     keepdims=True)
        candy_ref[t:t + 1, :] = it
        if t < T - 1:
            d = jnp.where(kk == it, _BIGF, d)
    candy_ref[T:T + 1, :] = y_ref[0]


def _refine_body(y_ref, x_ref, wc_ref, candy_ref,
                 close_ref, qst_ref, qp_ref, xlp_ref):
    xv = x_ref[...]
    xt = xv.T
    dsq_rows = []
    for t in range(T):
        wct = wc_ref[t].T
        diff = xt - wct
        dsq_rows.append(_exact_tree(diff * diff))
    dsq = jnp.concatenate(dsq_rows, axis=0)
    sd = dsq * lax.rsqrt(dsq)
    cidx = candy_ref[0:T, :]
    m = jnp.min(sd, axis=0, keepdims=True)
    idx = jnp.min(jnp.where(sd == m, cidx, _BIGI), axis=0,
                  keepdims=True)
    dmin = jnp.min(dsq, axis=0, keepdims=True)
    yi = y_ref[0]
    ind = (idx != yi).astype(jnp.float32)
    close_ref[0] = idx
    xlp_ref[0:1, 0:1, 0:1] = jnp.sum(ind * dmin).reshape(1, 1, 1)
    q = wc_ref[T] - xv
    qst_ref[...] = xv + q
    qp_ref[0:1, 0:1, 0:1] = jnp.sum(q * q).reshape(1, 1, 1)


def _final_body(y2_ref, qp_ref, xlp_ref, loss_ref, perp_ref):
    qsum = jnp.sum(qp_ref[...])
    xlsum = jnp.sum(xlp_ref[...])
    scale = 1.0 / (B * ED)
    loss_ref[0:1, 0:1] = ((1.0 + _COMMIT) * qsum * scale
                          - (1.0 + _DIVERGE) * xlsum * scale).reshape(1, 1)
    ycol = y2_ref[...]
    kk = lax.broadcasted_iota(jnp.int32, (1, K), 1)
    eq = (ycol == kk).astype(jnp.float32)
    counts = jnp.sum(eq, axis=0, keepdims=True)
    probs = counts * (1.0 / B)
    ent = jnp.sum(probs * jnp.log(probs + 1e-10))
    perp_ref[0:1, 0:1] = jnp.exp(-ent).reshape(1, 1)


_SC_NC = 2
_SC_NS = 16
_NW = _SC_NC * _SC_NS


@functools.cache
def _gather_rows_kernel(nrows):
    bpw = nrows // _NW
    mesh = plsc.VectorSubcoreMesh(core_axis_name="c", subcore_axis_name="s")

    @functools.partial(
        pl.kernel, mesh=mesh,
        out_type=jax.ShapeDtypeStruct((nrows, ED), jnp.float32),
        scratch_types=[
            pltpu.VMEM((bpw,), jnp.int32),
            pltpu.VMEM((bpw, ED), jnp.float32),
            pltpu.SemaphoreType.DMA,
        ],
    )
    def _gather_rows(table_hbm, idx_hbm, out_hbm, idx_v, rows_v, sem):
        wid = lax.axis_index("s") * _SC_NC + lax.axis_index("c")
        base = wid * bpw
        pltpu.sync_copy(idx_hbm.at[pl.ds(base, bpw)], idx_v)
        pltpu.async_copy(table_hbm.at[idx_v], rows_v, sem).wait()
        pltpu.sync_copy(rows_v, out_hbm.at[pl.ds(base, bpw)])

    return _gather_rows


def kernel(x, y, W):
    y32 = y.astype(jnp.int32)
    y3 = y32.reshape(NB, 1, BI)

    candy = pl.pallas_call(
        _coarse_body,
        grid=(NB,),
        in_specs=[
            pl.BlockSpec((1, 1, BI), lambda ib: (ib, 0, 0)),
            pl.BlockSpec((BI, ED), lambda ib: (ib, 0)),
            pl.BlockSpec((K, ED), lambda ib: (0, 0)),
        ],
        out_specs=pl.BlockSpec((T + 1, BI), lambda ib: (0, ib)),
        out_shape=jax.ShapeDtypeStruct((T + 1, B), jnp.int32),
    )(y3, x, W)

    wc = _gather_rows_kernel((T + 1) * B)(W, candy.reshape((T + 1) * B))
    wc = wc.reshape(T + 1, B, ED)

    close3, qst, qp, xlp = pl.pallas_call(
        _refine_body,
        grid=(NB,),
        in_specs=[
            pl.BlockSpec((1, 1, BI), lambda ib: (ib, 0, 0)),
            pl.BlockSpec((BI, ED), lambda ib: (ib, 0)),
            pl.BlockSpec((T + 1, BI, ED), lambda ib: (0, ib, 0)),
            pl.BlockSpec((T + 1, BI), lambda ib: (0, ib)),
        ],
        out_specs=[
            pl.BlockSpec((1, 1, BI), lambda ib: (ib, 0, 0)),
            pl.BlockSpec((BI, ED), lambda ib: (ib, 0)),
            pl.BlockSpec((1, 1, 1), lambda ib: (ib, 0, 0)),
            pl.BlockSpec((1, 1, 1), lambda ib: (ib, 0, 0)),
        ],
        out_shape=[
            jax.ShapeDtypeStruct((NB, 1, BI), jnp.int32),
            jax.ShapeDtypeStruct((B, ED), jnp.float32),
            jax.ShapeDtypeStruct((NB, 1, 1), jnp.float32),
            jax.ShapeDtypeStruct((NB, 1, 1), jnp.float32),
        ],
    )(y3, x, wc, candy)

    loss, perp = pl.pallas_call(
        _final_body,
        out_shape=[
            jax.ShapeDtypeStruct((1, 1), jnp.float32),
            jax.ShapeDtypeStruct((1, 1), jnp.float32),
        ],
    )(y32.reshape(B, 1), qp, xlp)

    return (loss.reshape(()), qst, perp.reshape(()), close3.reshape(B, 1))


# speedup vs baseline: 5.8716x; 1.0237x over previous
import functools

import jax
import jax.numpy as jnp
from jax import lax
from jax.experimental import pallas as pl
from jax.experimental.pallas import tpu as pltpu
from jax.experimental.pallas import tpu_sc as plsc

K = 1024
ED = 256
B = 1024
T = 6
BI = 128
NB = B // BI

_COMMIT = 0.25
_DIVERGE = 0.1
_BIGF = 3.0e38
_BIGI = 2 ** 30


def _exact_tree(sq):
    L = sq.shape[1]
    rs = sq.reshape(2, 16, 8, L)
    acc = rs[:, 0]
    for j in range(1, 16):
        acc = acc + rs[:, j]
    v = ((acc[:, 0] + acc[:, 4]) + (acc[:, 2] + acc[:, 6])) + (
        (acc[:, 1] + acc[:, 5]) + (acc[:, 3] + acc[:, 7]))
    return v[0:1] + v[1:2]


def _coarse_body(y_ref, x_ref, w_ref, candy_ref):
    xt = x_ref[...].T
    w = w_ref[...]
    g = lax.dot_general(w, xt, (((1,), (0,)), ((), ())),
                        precision=lax.Precision.HIGHEST,
                        preferred_element_type=jnp.float32)
    wn = jnp.sum(w * w, axis=1, keepdims=True)
    d = wn - (g + g)
    kk = lax.broadcasted_iota(jnp.int32, (K, BI), 0)
    for t in range(T):
        m = jnp.min(d, axis=0, keepdims=True)
        it = jnp.min(jnp.where(d == m, kk, _BIGI), axis=0,
                     keepdims=True)
        candy_ref[t:t + 1, :] = it
        if t < T - 1:
            d = jnp.where(kk == it, _BIGF, d)
    candy_ref[T:T + 1, :] = y_ref[0]


def _refine_body(y_ref, yc_ref, x_ref, wc_ref, candy_ref,
                 close_ref, qst_ref, loss_ref, perp_ref,
                 hist_ref, acc_ref):
    ib = pl.program_id(0)

    @pl.when(ib == 0)
    def _init():
        hist_ref[...] = jnp.zeros((1, K), jnp.float32)
        acc_ref[0] = 0.0
        acc_ref[1] = 0.0

    xv = x_ref[...]
    xt = xv.T
    dsq_rows = []
    for t in range(T):
        wct = wc_ref[t].T
        diff = xt - wct
        dsq_rows.append(_exact_tree(diff * diff))
    dsq = jnp.concatenate(dsq_rows, axis=0)
    sd = dsq * lax.rsqrt(dsq)
    cidx = candy_ref[0:T, :]
    m = jnp.min(sd, axis=0, keepdims=True)
    idx = jnp.min(jnp.where(sd == m, cidx, _BIGI), axis=0,
                  keepdims=True)
    dmin = jnp.min(dsq, axis=0, keepdims=True)
    yi = y_ref[0]
    ind = (idx != yi).astype(jnp.float32)
    close_ref[0] = idx
    q = wc_ref[T] - xv
    qst_ref[...] = xv + q
    acc_ref[0] = acc_ref[0] + jnp.sum(q * q)
    acc_ref[1] = acc_ref[1] + jnp.sum(ind * dmin)
    kk2 = lax.broadcasted_iota(jnp.int32, (1, K), 1)
    eq = (yc_ref[...] == kk2).astype(jnp.float32)
    hist_ref[...] = hist_ref[...] + jnp.sum(eq, axis=0, keepdims=True)

    @pl.when(ib == NB - 1)
    def _fin():
        scale = 1.0 / (B * ED)
        loss_ref[0:1, 0:1] = ((1.0 + _COMMIT) * acc_ref[0] * scale
                              - (1.0 + _DIVERGE) * acc_ref[1] * scale
                              ).reshape(1, 1)
        probs = hist_ref[...] * (1.0 / B)
        ent = jnp.sum(probs * jnp.log(probs + 1e-10))
        perp_ref[0:1, 0:1] = jnp.exp(-ent).reshape(1, 1)


_SC_NC = 2
_SC_NS = 16
_NW = _SC_NC * _SC_NS


_NCH = 4


@functools.cache
def _gather_rows_kernel(nrows):
    bpw = nrows // _NW
    cpw = bpw // _NCH
    mesh = plsc.VectorSubcoreMesh(core_axis_name="c", subcore_axis_name="s")

    @functools.partial(
        pl.kernel, mesh=mesh,
        out_type=jax.ShapeDtypeStruct((nrows, ED), jnp.float32),
        scratch_types=[
            pltpu.VMEM((bpw,), jnp.int32),
            pltpu.VMEM((_NCH, cpw, ED), jnp.float32),
        ] + [pltpu.SemaphoreType.DMA] * _NCH,
    )
    def _gather_rows(table_hbm, idx_hbm, out_hbm, idx_v, rows_v, *sems):
        wid = lax.axis_index("s") * _SC_NC + lax.axis_index("c")
        base = wid * bpw
        pltpu.sync_copy(idx_hbm.at[pl.ds(base, bpw)], idx_v)
        cps = [pltpu.async_copy(table_hbm.at[idx_v.at[pl.ds(c * cpw, cpw)]],
                                rows_v.at[c], sems[c])
               for c in range(_NCH)]
        for c in range(_NCH):
            cps[c].wait()
            pltpu.sync_copy(rows_v.at[c], out_hbm.at[pl.ds(base + c * cpw, cpw)])

    return _gather_rows


def kernel(x, y, W):
    y32 = y.astype(jnp.int32)
    y3 = y32.reshape(NB, 1, BI)

    candy = pl.pallas_call(
        _coarse_body,
        grid=(NB,),
        in_specs=[
            pl.BlockSpec((1, 1, BI), lambda ib: (ib, 0, 0)),
            pl.BlockSpec((BI, ED), lambda ib: (ib, 0)),
            pl.BlockSpec((K, ED), lambda ib: (0, 0)),
        ],
        out_specs=pl.BlockSpec((T + 1, BI), lambda ib: (0, ib)),
        out_shape=jax.ShapeDtypeStruct((T + 1, B), jnp.int32),
    )(y3, x, W)

    wc = _gather_rows_kernel((T + 1) * B)(W, candy.reshape((T + 1) * B))
    wc = wc.reshape(T + 1, B, ED)

    close3, qst, loss, perp = pl.pallas_call(
        _refine_body,
        grid=(NB,),
        in_specs=[
            pl.BlockSpec((1, 1, BI), lambda ib: (ib, 0, 0)),
            pl.BlockSpec((BI, 1), lambda ib: (ib, 0)),
            pl.BlockSpec((BI, ED), lambda ib: (ib, 0)),
            pl.BlockSpec((T + 1, BI, ED), lambda ib: (0, ib, 0)),
            pl.BlockSpec((T + 1, BI), lambda ib: (0, ib)),
        ],
        out_specs=[
            pl.BlockSpec((1, 1, BI), lambda ib: (ib, 0, 0)),
            pl.BlockSpec((BI, ED), lambda ib: (ib, 0)),
            pl.BlockSpec((1, 1), lambda ib: (0, 0)),
            pl.BlockSpec((1, 1), lambda ib: (0, 0)),
        ],
        out_shape=[
            jax.ShapeDtypeStruct((NB, 1, BI), jnp.int32),
            jax.ShapeDtypeStruct((B, ED), jnp.float32),
            jax.ShapeDtypeStruct((1, 1), jnp.float32),
            jax.ShapeDtypeStruct((1, 1), jnp.float32),
        ],
        scratch_shapes=[
            pltpu.VMEM((1, K), jnp.float32),
            pltpu.SMEM((2,), jnp.float32),
        ],
    )(y3, y32.reshape(B, 1), x, wc, candy)

    return (loss.reshape(()), qst, perp.reshape(()), close3.reshape(B, 1))


# speedup vs baseline: 6.7301x; 1.1462x over previous
import functools

import jax
import jax.numpy as jnp
from jax import lax
from jax.experimental import pallas as pl
from jax.experimental.pallas import tpu as pltpu
from jax.experimental.pallas import tpu_sc as plsc

K = 1024
ED = 256
B = 1024
T = 6
BI = 128
NB = B // BI

_COMMIT = 0.25
_DIVERGE = 0.1
_BIGF = 3.0e38
_BIGI = 2 ** 30


def _exact_tree(sq):
    L = sq.shape[1]
    rs = sq.reshape(2, 16, 8, L)
    acc = rs[:, 0]
    for j in range(1, 16):
        acc = acc + rs[:, j]
    v = ((acc[:, 0] + acc[:, 4]) + (acc[:, 2] + acc[:, 6])) + (
        (acc[:, 1] + acc[:, 5]) + (acc[:, 3] + acc[:, 7]))
    return v[0:1] + v[1:2]


def _coarse_body(y_ref, x_ref, w_ref, candy_ref):
    xt = x_ref[...].T
    w = w_ref[...]
    g = lax.dot_general(w, xt, (((1,), (0,)), ((), ())),
                        precision=lax.Precision.HIGHEST,
                        preferred_element_type=jnp.float32)
    wn = jnp.sum(w * w, axis=1, keepdims=True)
    d = wn - (g + g)
    nc = K // 128
    dc = d.reshape(nc, 128, BI)
    kkc = lax.broadcasted_iota(jnp.int32, (nc, 128, BI), 1)
    cbase = lax.broadcasted_iota(jnp.int32, (nc, 1, BI), 0) * 128
    vals, gidx = [], []
    for t in range(3):
        m8 = jnp.min(dc, axis=1, keepdims=True)
        i8 = jnp.min(jnp.where(dc == m8, kkc, _BIGI), axis=1,
                     keepdims=True)
        vals.append(m8.reshape(nc, BI))
        gidx.append((i8 + cbase).reshape(nc, BI))
        if t < 2:
            dc = jnp.where(kkc == i8, _BIGF, dc)
    va = jnp.concatenate(vals, axis=0)
    ga = jnp.concatenate(gidx, axis=0)
    for t in range(T):
        m = jnp.min(va, axis=0, keepdims=True)
        it = jnp.min(jnp.where(va == m, ga, _BIGI), axis=0,
                     keepdims=True)
        candy_ref[t:t + 1, :] = it
        if t < T - 1:
            va = jnp.where(ga == it, _BIGF, va)
    candy_ref[T:T + 1, :] = y_ref[0]


def _refine_body(y_ref, yc_ref, x_ref, wc_ref, candy_ref,
                 close_ref, qst_ref, loss_ref, perp_ref,
                 hist_ref, acc_ref):
    ib = pl.program_id(0)

    @pl.when(ib == 0)
    def _init():
        hist_ref[...] = jnp.zeros((1, K), jnp.float32)
        acc_ref[0] = 0.0
        acc_ref[1] = 0.0

    xv = x_ref[...]
    xt = xv.T
    dsq_rows = []
    for t in range(T):
        wct = wc_ref[t].T
        diff = xt - wct
        dsq_rows.append(_exact_tree(diff * diff))
    dsq = jnp.concatenate(dsq_rows, axis=0)
    sd = dsq * lax.rsqrt(dsq)
    cidx = candy_ref[0:T, :]
    m = jnp.min(sd, axis=0, keepdims=True)
    idx = jnp.min(jnp.where(sd == m, cidx, _BIGI), axis=0,
                  keepdims=True)
    dmin = jnp.min(dsq, axis=0, keepdims=True)
    yi = y_ref[0]
    ind = (idx != yi).astype(jnp.float32)
    close_ref[0] = idx
    q = wc_ref[T] - xv
    qst_ref[...] = xv + q
    acc_ref[0] = acc_ref[0] + jnp.sum(q * q)
    acc_ref[1] = acc_ref[1] + jnp.sum(ind * dmin)
    kk2 = lax.broadcasted_iota(jnp.int32, (1, K), 1)
    eq = (yc_ref[...] == kk2).astype(jnp.float32)
    hist_ref[...] = hist_ref[...] + jnp.sum(eq, axis=0, keepdims=True)

    @pl.when(ib == NB - 1)
    def _fin():
        scale = 1.0 / (B * ED)
        loss_ref[0:1, 0:1] = ((1.0 + _COMMIT) * acc_ref[0] * scale
                              - (1.0 + _DIVERGE) * acc_ref[1] * scale
                              ).reshape(1, 1)
        probs = hist_ref[...] * (1.0 / B)
        ent = jnp.sum(probs * jnp.log(probs + 1e-10))
        perp_ref[0:1, 0:1] = jnp.exp(-ent).reshape(1, 1)


_SC_NC = 2
_SC_NS = 16
_NW = _SC_NC * _SC_NS


_NCH = 4


@functools.cache
def _gather_rows_kernel(nrows):
    bpw = nrows // _NW
    cpw = bpw // _NCH
    mesh = plsc.VectorSubcoreMesh(core_axis_name="c", subcore_axis_name="s")

    @functools.partial(
        pl.kernel, mesh=mesh,
        out_type=jax.ShapeDtypeStruct((nrows, ED), jnp.float32),
        scratch_types=[
            pltpu.VMEM((bpw,), jnp.int32),
            pltpu.VMEM((_NCH, cpw, ED), jnp.float32),
        ] + [pltpu.SemaphoreType.DMA] * _NCH,
    )
    def _gather_rows(table_hbm, idx_hbm, out_hbm, idx_v, rows_v, *sems):
        wid = lax.axis_index("s") * _SC_NC + lax.axis_index("c")
        base = wid * bpw
        pltpu.sync_copy(idx_hbm.at[pl.ds(base, bpw)], idx_v)
        cps = [pltpu.async_copy(table_hbm.at[idx_v.at[pl.ds(c * cpw, cpw)]],
                                rows_v.at[c], sems[c])
               for c in range(_NCH)]
        for c in range(_NCH):
            cps[c].wait()
            pltpu.sync_copy(rows_v.at[c], out_hbm.at[pl.ds(base + c * cpw, cpw)])

    return _gather_rows


def kernel(x, y, W):
    y32 = y.astype(jnp.int32)
    y3 = y32.reshape(NB, 1, BI)

    candy = pl.pallas_call(
        _coarse_body,
        grid=(NB,),
        in_specs=[
            pl.BlockSpec((1, 1, BI), lambda ib: (ib, 0, 0)),
            pl.BlockSpec((BI, ED), lambda ib: (ib, 0)),
            pl.BlockSpec((K, ED), lambda ib: (0, 0)),
        ],
        out_specs=pl.BlockSpec((T + 1, BI), lambda ib: (0, ib)),
        out_shape=jax.ShapeDtypeStruct((T + 1, B), jnp.int32),
    )(y3, x, W)

    wc = _gather_rows_kernel((T + 1) * B)(W, candy.reshape((T + 1) * B))
    wc = wc.reshape(T + 1, B, ED)

    close3, qst, loss, perp = pl.pallas_call(
        _refine_body,
        grid=(NB,),
        in_specs=[
            pl.BlockSpec((1, 1, BI), lambda ib: (ib, 0, 0)),
            pl.BlockSpec((BI, 1), lambda ib: (ib, 0)),
            pl.BlockSpec((BI, ED), lambda ib: (ib, 0)),
            pl.BlockSpec((T + 1, BI, ED), lambda ib: (0, ib, 0)),
            pl.BlockSpec((T + 1, BI), lambda ib: (0, ib)),
        ],
        out_specs=[
            pl.BlockSpec((1, 1, BI), lambda ib: (ib, 0, 0)),
            pl.BlockSpec((BI, ED), lambda ib: (ib, 0)),
            pl.BlockSpec((1, 1), lambda ib: (0, 0)),
            pl.BlockSpec((1, 1), lambda ib: (0, 0)),
        ],
        out_shape=[
            jax.ShapeDtypeStruct((NB, 1, BI), jnp.int32),
            jax.ShapeDtypeStruct((B, ED), jnp.float32),
            jax.ShapeDtypeStruct((1, 1), jnp.float32),
            jax.ShapeDtypeStruct((1, 1), jnp.float32),
        ],
        scratch_shapes=[
            pltpu.VMEM((1, K), jnp.float32),
            pltpu.SMEM((2,), jnp.float32),
        ],
    )(y3, y32.reshape(B, 1), x, wc, candy)

    return (loss.reshape(()), qst, perp.reshape(()), close3.reshape(B, 1))
